# baseline (speedup 1.0000x reference)
import functools

import jax
import jax.numpy as jnp
from jax import lax
from jax.experimental import pallas as pl
from jax.experimental.pallas import tpu as pltpu
from jax.experimental.pallas import tpu_sc as plsc

B, N, D, E, FE = 2, 10000, 256, 160000, 16
H = D // 2
NT = 16
EPT = E // NT
CH = 80
NFULL = EPT // CH
RPT = 640
STG = 80


def _gelu_exact(x):
    return 0.5 * x * (1.0 + lax.erf(x * 0.7071067811865476))


def _gelu_sc(x):
    u = x * x
    t2 = x * (1.5957691216057308 + 0.0713548703591064 * u)
    e = jnp.exp(jnp.minimum(t2, 40.0))
    return x - x / (e + 1.0)


def _pre_body(h_ref, w_ref, b_ref, out0_ref, out1_ref):
    y = jnp.dot(h_ref[...], w_ref[...], preferred_element_type=jnp.float32)
    y = y + b_ref[...]
    out0_ref[...] = y[:, :H]
    out1_ref[...] = y[:, H:]


def _pre_call(h_flat, w, b):
    blk = 400
    grid = (B * N) // blk
    return pl.pallas_call(
        _pre_body,
        grid=(grid,),
        in_specs=[
            pl.BlockSpec((blk, D), lambda i: (i, 0)),
            pl.BlockSpec((D, D), lambda i: (0, 0)),
            pl.BlockSpec((1, D), lambda i: (0, 0)),
        ],
        out_specs=[pl.BlockSpec((blk, H), lambda i: (i, 0)),
                   pl.BlockSpec((blk, H), lambda i: (i, 0))],
        out_shape=[jax.ShapeDtypeStruct((B * N, H), jnp.float32),
                   jax.ShapeDtypeStruct((B * N, H), jnp.float32)],
    )(h_flat, w, b)


def _ea_body(ea_ref, w_ref, out0_ref, out1_ref):
    y = jnp.dot(ea_ref[...], w_ref[...], preferred_element_type=jnp.float32)
    out0_ref[...] = y[:, :H]
    out1_ref[...] = y[:, H:]


def _ea_call(edge_attr, w):
    blk = 2000
    grid = E // blk
    return pl.pallas_call(
        _ea_body,
        grid=(grid,),
        in_specs=[
            pl.BlockSpec((blk, FE), lambda i: (i, 0)),
            pl.BlockSpec((FE, D), lambda i: (0, 0)),
        ],
        out_specs=[pl.BlockSpec((blk, H), lambda i: (i, 0)),
                   pl.BlockSpec((blk, H), lambda i: (i, 0))],
        out_shape=[jax.ShapeDtypeStruct((E, H), jnp.float32),
                   jax.ShapeDtypeStruct((E, H), jnp.float32)],
    )(edge_attr, w)


def _tile_rows(s, do):
    @pl.when(s < NT - 1)
    def _():
        for j in range(RPT // STG):
            do(s * RPT + j * STG)

    @pl.when(s == NT - 1)
    def _():
        for j in range((N - (NT - 1) * RPT) // STG):
            do((NT - 1) * RPT + j * STG)


def _sc_edge_kernel(pre0, pre1, ea0, ea1, gsrc, dstv, zrow,
                    out0, out1,
                    idx_s0, idx_d0, idx_s1, idx_d1, rows0, rows1, eav,
                    agg_sh, semg0, semg1, seme, semi0, semi1, semS0, semS1):
    c = lax.axis_index("c")
    s = lax.axis_index("s")
    ebase = s * EPT

    for r in range(2):
        pre_t = pre0 if r == 0 else pre1
        ea_t = ea0 if r == 0 else ea1
        out_t = out0 if r == 0 else out1

        pltpu.sync_copy(zrow, rows0)
        _tile_rows(s, lambda o: pltpu.sync_copy(
            rows0, agg_sh.at[pl.ds(o, STG)]))
        plsc.subcore_barrier()

        def start_idx(i, i_s, i_d, semi):
            e0 = ebase + i * CH
            pltpu.async_copy(gsrc.at[pl.ds(c * E + e0, CH)], i_s, semi)
            pltpu.async_copy(dstv.at[pl.ds(e0, CH)], i_d, semi)

        def start_g(i_s, i_d, rw, semg, semi):
            pltpu.make_async_copy(gsrc.at[pl.ds(0, CH)], i_s, semi).wait()
            pltpu.make_async_copy(dstv.at[pl.ds(0, CH)], i_d, semi).wait()
            pltpu.async_copy(pre_t.at[i_s], rw, semg)

        def start_ea(i):
            e0 = ebase + i * CH
            pltpu.async_copy(ea_t.at[pl.ds(e0, CH)], eav, seme)

        def finish(i_s, i_d, rw, semg):
            pltpu.make_async_copy(pre_t.at[i_s], rw, semg).wait()
            pltpu.make_async_copy(ea_t.at[pl.ds(0, CH)], eav, seme).wait()

            @plsc.parallel_loop(0, CH, step=1, unroll=2)
            def _row(j):
                for k in range(H // 16):
                    x = rw[j, pl.ds(k * 16, 16)] + eav[j, pl.ds(k * 16, 16)]
                    rw[j, pl.ds(k * 16, 16)] = _gelu_sc(x)

        def scatter_start(i_d, rw, sems):
            pltpu.async_copy(rw, agg_sh.at[i_d], sems, add=True)

        def scatter_wait(rw, sems):
            pltpu.make_async_copy(rw, agg_sh.at[pl.ds(0, CH)], sems).wait()

        start_idx(0, idx_s0, idx_d0, semi0)
        start_g(idx_s0, idx_d0, rows0, semg0, semi0)
        start_ea(0)
        start_idx(1, idx_s1, idx_d1, semi1)
        pltpu.async_copy(zrow, rows1, semS1)

        def pair(k, carry):
            a = 2 * k
            scatter_wait(rows1, semS1)
            start_g(idx_s1, idx_d1, rows1, semg1, semi1)
            finish(idx_s0, idx_d0, rows0, semg0)
            start_idx(a + 2, idx_s0, idx_d0, semi0)
            start_ea(a + 1)
            scatter_start(idx_d0, rows0, semS0)
            finish(idx_s1, idx_d1, rows1, semg1)
            start_idx(a + 3, idx_s1, idx_d1, semi1)
            start_ea(a + 2)
            scatter_wait(rows0, semS0)
            start_g(idx_s0, idx_d0, rows0, semg0, semi0)
            scatter_start(idx_d1, rows1, semS1)
            return carry

        lax.fori_loop(0, (NFULL - 1) // 2, pair, 0)
        finish(idx_s0, idx_d0, rows0, semg0)
        pltpu.sync_copy(rows0, agg_sh.at[idx_d0], add=True)
        scatter_wait(rows1, semS1)
        pltpu.make_async_copy(gsrc.at[pl.ds(0, CH)], idx_s1, semi1).wait()
        pltpu.make_async_copy(dstv.at[pl.ds(0, CH)], idx_d1, semi1).wait()

        plsc.subcore_barrier()

        def _out(o):
            pltpu.sync_copy(agg_sh.at[pl.ds(o, STG)], rows0)
            pltpu.sync_copy(rows0, out_t.at[c, pl.ds(o, STG)])

        _tile_rows(s, _out)


def _sc_edge_call(pre0, pre1, ea0, ea1, gsrc, dstv):
    zrow = jnp.zeros((STG, H), jnp.float32)
    mesh = plsc.VectorSubcoreMesh(core_axis_name="c", subcore_axis_name="s")
    fn = functools.partial(
        pl.kernel,
        mesh=mesh,
        out_type=[
            jax.ShapeDtypeStruct((B, N, H), jnp.float32),
            jax.ShapeDtypeStruct((B, N, H), jnp.float32),
        ],
        scratch_types=[
            pltpu.VMEM((CH,), jnp.int32),
            pltpu.VMEM((CH,), jnp.int32),
            pltpu.VMEM((CH,), jnp.int32),
            pltpu.VMEM((CH,), jnp.int32),
            pltpu.VMEM((CH, H), jnp.float32),
            pltpu.VMEM((CH, H), jnp.float32),
            pltpu.VMEM((CH, H), jnp.float32),
            pltpu.VMEM_SHARED((N, H), jnp.float32),
            pltpu.SemaphoreType.DMA,
            pltpu.SemaphoreType.DMA,
            pltpu.SemaphoreType.DMA,
            pltpu.SemaphoreType.DMA,
            pltpu.SemaphoreType.DMA,
            pltpu.SemaphoreType.DMA,
            pltpu.SemaphoreType.DMA,
        ],
    )(_sc_edge_kernel)
    return fn(pre0, pre1, ea0, ea1, gsrc, dstv, zrow)


EPT_D = E // (2 * NT)
NFULL_D = EPT_D // CH
TAIL_D = EPT_D - NFULL_D * CH


def _sc_deg_kernel(dstv, zcnt, ones_hbm, cnt0_out, cnt1_out,
                   idx_d, idx_t, ones_v, stg, stg16, cnt_sh):
    c = lax.axis_index("c")
    s = lax.axis_index("s")
    ebase = c * (E // 2) + s * EPT_D

    pltpu.sync_copy(ones_hbm, ones_v)
    pltpu.sync_copy(zcnt, stg)
    _tile_rows(s, lambda o: pltpu.sync_copy(
        stg, cnt_sh.at[pl.ds(o, STG)]))
    plsc.subcore_barrier()

    def chunk(i, carry):
        e0 = ebase + i * CH
        pltpu.sync_copy(dstv.at[pl.ds(e0, CH)], idx_d)
        pltpu.sync_copy(ones_v, cnt_sh.at[idx_d], add=True)
        return carry

    lax.fori_loop(0, NFULL_D, chunk, 0)
    e0t = ebase + NFULL_D * CH
    pltpu.sync_copy(dstv.at[pl.ds(e0t, TAIL_D)], idx_t)
    pltpu.sync_copy(ones_v.at[pl.ds(0, TAIL_D)],
                    cnt_sh.at[idx_t], add=True)
    plsc.subcore_barrier()

    def _out(o):
        pltpu.sync_copy(cnt_sh.at[pl.ds(o, STG)], stg)

        def row(j, cc):
            stg16[j, pl.ds(0, 16)] = stg[j, pl.ds(0, 16)]
            return cc

        lax.fori_loop(0, STG, row, 0)

        @pl.when(c == 0)
        def _():
            pltpu.sync_copy(stg16, cnt0_out.at[pl.ds(o, STG)])

        @pl.when(c == 1)
        def _():
            pltpu.sync_copy(stg16, cnt1_out.at[pl.ds(o, STG)])

    _tile_rows(s, _out)


def _sc_deg_call(dstv):
    zcnt = jnp.zeros((STG, H), jnp.float32)
    ones_hbm = jnp.ones((CH, H), jnp.float32)
    mesh = plsc.VectorSubcoreMesh(core_axis_name="c", subcore_axis_name="s")
    fn = functools.partial(
        pl.kernel,
        mesh=mesh,
        out_type=[jax.ShapeDtypeStruct((N, 16), jnp.float32),
                  jax.ShapeDtypeStruct((N, 16), jnp.float32)],
        scratch_types=[
            pltpu.VMEM((CH,), jnp.int32),
            pltpu.VMEM((TAIL_D,), jnp.int32),
            pltpu.VMEM((CH, H), jnp.float32),
            pltpu.VMEM((STG, H), jnp.float32),
            pltpu.VMEM((STG, 16), jnp.float32),
            pltpu.VMEM_SHARED((N, H), jnp.float32),
        ],
    )(_sc_deg_kernel)
    return fn(dstv, zcnt, ones_hbm)


def _upd_body(h_ref, s0_ref, s1_ref, cnt0_ref, cnt1_ref,
              w2m0_ref, w2m1_ref, b2m_ref,
              w1uh_ref, w1ua_ref, b1u_ref, w2u_ref, b2u_ref, g_ref, be_ref,
              out_ref):
    hb = h_ref[...]
    agg = jnp.dot(s0_ref[...], w2m0_ref[...], preferred_element_type=jnp.float32)
    agg = agg + jnp.dot(s1_ref[...], w2m1_ref[...], preferred_element_type=jnp.float32)
    deg = cnt0_ref[:, 0:1] + cnt1_ref[:, 0:1]
    agg = agg + deg * b2m_ref[...]
    u = jnp.dot(hb, w1uh_ref[...], preferred_element_type=jnp.float32)
    u = u + jnp.dot(agg, w1ua_ref[...], preferred_element_type=jnp.float32)
    u = u + b1u_ref[...]
    g2 = _gelu_exact(u)
    dh = jnp.dot(g2, w2u_ref[...], preferred_element_type=jnp.float32) + b2u_ref[...]
    y = hb + dh
    mu = jnp.mean(y, axis=-1, keepdims=True)
    var = jnp.mean((y - mu) ** 2, axis=-1, keepdims=True)
    out_ref[...] = (y - mu) * lax.rsqrt(var + 1e-5) * g_ref[...] + be_ref[...]


def _upd_call(h_flat, s0, s1, cnt0, cnt1, w2m, b2m, w1u, b1u, w2u, b2u,
              gamma, beta):
    blk = 400
    grid = (B * N) // blk
    nblk = N // blk
    full = lambda shape: pl.BlockSpec(shape, lambda i: tuple(0 for _ in shape))
    return pl.pallas_call(
        _upd_body,
        grid=(grid,),
        in_specs=[
            pl.BlockSpec((blk, D), lambda i: (i, 0)),
            pl.BlockSpec((blk, H), lambda i: (i, 0)),
            pl.BlockSpec((blk, H), lambda i: (i, 0)),
            pl.BlockSpec((blk, 16), lambda i: (i % nblk, 0)),
            pl.BlockSpec((blk, 16), lambda i: (i % nblk, 0)),
            full((H, D)), full((H, D)), full((1, D)),
            full((D, D)), full((D, D)), full((1, D)),
            full((D, D)), full((1, D)), full((1, D)), full((1, D)),
        ],
        out_specs=pl.BlockSpec((blk, D), lambda i: (i, 0)),
        out_shape=jax.ShapeDtypeStruct((B * N, D), jnp.float32),
    )(h_flat, s0, s1, cnt0, cnt1, w2m[:H], w2m[H:], b2m.reshape(1, D),
      w1u[:D], w1u[D:], b1u.reshape(1, D), w2u, b2u.reshape(1, D),
      gamma.reshape(1, D), beta.reshape(1, D))


def kernel(h, edge_index, edge_attr, W1m, b1m, W2m, b2m,
           W1u, b1u, W2u, b2u, gamma, beta):
    h_flat = h.reshape(B * N, D)
    src = edge_index[0]
    dst = edge_index[1]
    offs = (jnp.arange(B, dtype=jnp.int32) * N)[:, None]
    pad = jnp.zeros((CH,), jnp.int32)
    gsrc = jnp.concatenate([(src[None, :] + offs).reshape(-1), pad])
    dst_p = jnp.concatenate([dst, pad])

    cnt0, cnt1 = _sc_deg_call(dst)
    pre0, pre1 = _pre_call(h_flat, W1m[:D], b1m.reshape(1, D))
    ea0, ea1 = _ea_call(edge_attr, W1m[D:])

    s0, s1 = _sc_edge_call(pre0, pre1, ea0, ea1, gsrc, dst_p)

    out = _upd_call(h_flat, s0.reshape(B * N, H), s1.reshape(B * N, H),
                    cnt0, cnt1, W2m, b2m, W1u, b1u, W2u, b2u,
                    gamma, beta)
    return out.reshape(B, N, D)


# speedup vs baseline: 1.0578x; 1.0578x over previous
import functools

import jax
import jax.numpy as jnp
from jax import lax
from jax.experimental import pallas as pl
from jax.experimental.pallas import tpu as pltpu
from jax.experimental.pallas import tpu_sc as plsc

B, N, D, E, FE = 2, 10000, 256, 160000, 16
H = D // 2
NT = 16
EPT = E // NT
CH = 80
NFULL = EPT // CH
RPT = 640
STG = 80


def _gelu_exact(x):
    return 0.5 * x * (1.0 + lax.erf(x * 0.7071067811865476))


def _gelu_sc(x):
    u = x * x
    t2 = x * (1.5957691216057308 + 0.0713548703591064 * u)
    e = jnp.exp(jnp.minimum(t2, 40.0))
    return x - x / (e + 1.0)


def _pre_body(h_ref, w_ref, b_ref, out0_ref, out1_ref):
    y = jnp.dot(h_ref[...], w_ref[...], preferred_element_type=jnp.float32)
    y = y + b_ref[...]
    out0_ref[...] = y[:, :H]
    out1_ref[...] = y[:, H:]


def _pre_call(h_flat, w, b):
    blk = 400
    grid = (B * N) // blk
    return pl.pallas_call(
        _pre_body,
        grid=(grid,),
        in_specs=[
            pl.BlockSpec((blk, D), lambda i: (i, 0)),
            pl.BlockSpec((D, D), lambda i: (0, 0)),
            pl.BlockSpec((1, D), lambda i: (0, 0)),
        ],
        out_specs=[pl.BlockSpec((blk, H), lambda i: (i, 0)),
                   pl.BlockSpec((blk, H), lambda i: (i, 0))],
        out_shape=[jax.ShapeDtypeStruct((B * N, H), jnp.float32),
                   jax.ShapeDtypeStruct((B * N, H), jnp.float32)],
    )(h_flat, w, b)


def _ea_body(ea_ref, w_ref, out0_ref, out1_ref):
    y = jnp.dot(ea_ref[...], w_ref[...], preferred_element_type=jnp.float32)
    out0_ref[...] = y[:, :H]
    out1_ref[...] = y[:, H:]


def _ea_call(edge_attr, w):
    blk = 2000
    grid = E // blk
    return pl.pallas_call(
        _ea_body,
        grid=(grid,),
        in_specs=[
            pl.BlockSpec((blk, FE), lambda i: (i, 0)),
            pl.BlockSpec((FE, D), lambda i: (0, 0)),
        ],
        out_specs=[pl.BlockSpec((blk, H), lambda i: (i, 0)),
                   pl.BlockSpec((blk, H), lambda i: (i, 0))],
        out_shape=[jax.ShapeDtypeStruct((E, H), jnp.float32),
                   jax.ShapeDtypeStruct((E, H), jnp.float32)],
    )(edge_attr, w)


def _tile_rows(s, do):
    @pl.when(s < NT - 1)
    def _():
        for j in range(RPT // STG):
            do(s * RPT + j * STG)

    @pl.when(s == NT - 1)
    def _():
        for j in range((N - (NT - 1) * RPT) // STG):
            do((NT - 1) * RPT + j * STG)


def _sc_edge_kernel(pre0, pre1, ea0, ea1, gsrc, dstv, zrow,
                    out0, out1,
                    idx_s0, idx_d0, idx_s1, idx_d1, rows0, rows1, eav,
                    agg_sh, semg0, semg1, seme, semi0, semi1):
    c = lax.axis_index("c")
    s = lax.axis_index("s")
    ebase = s * EPT

    for r in range(2):
        pre_t = pre0 if r == 0 else pre1
        ea_t = ea0 if r == 0 else ea1
        out_t = out0 if r == 0 else out1

        pltpu.sync_copy(zrow, rows0)
        _tile_rows(s, lambda o: pltpu.sync_copy(
            rows0, agg_sh.at[pl.ds(o, STG)]))
        plsc.subcore_barrier()

        def start_idx(i, i_s, i_d, semi):
            e0 = ebase + i * CH
            pltpu.async_copy(gsrc.at[pl.ds(c * E + e0, CH)], i_s, semi)
            pltpu.async_copy(dstv.at[pl.ds(e0, CH)], i_d, semi)

        def start_g(i_s, i_d, rw, semg, semi):
            pltpu.make_async_copy(gsrc.at[pl.ds(0, CH)], i_s, semi).wait()
            pltpu.make_async_copy(dstv.at[pl.ds(0, CH)], i_d, semi).wait()
            pltpu.async_copy(pre_t.at[i_s], rw, semg)

        def start_ea(i):
            e0 = ebase + i * CH
            pltpu.async_copy(ea_t.at[pl.ds(e0, CH)], eav, seme)

        def finish(i_s, i_d, rw, semg):
            pltpu.make_async_copy(pre_t.at[i_s], rw, semg).wait()
            pltpu.make_async_copy(ea_t.at[pl.ds(0, CH)], eav, seme).wait()

            @plsc.parallel_loop(0, CH, step=1, unroll=2)
            def _row(j):
                for k in range(H // 16):
                    x = rw[j, pl.ds(k * 16, 16)] + eav[j, pl.ds(k * 16, 16)]
                    rw[j, pl.ds(k * 16, 16)] = _gelu_sc(x)

        def scatter(i_d, rw):
            pltpu.sync_copy(rw, agg_sh.at[i_d], add=True)

        start_idx(0, idx_s0, idx_d0, semi0)
        start_g(idx_s0, idx_d0, rows0, semg0, semi0)
        start_ea(0)
        start_idx(1, idx_s1, idx_d1, semi1)

        def pair(k, carry):
            a = 2 * k
            start_g(idx_s1, idx_d1, rows1, semg1, semi1)
            finish(idx_s0, idx_d0, rows0, semg0)
            start_idx(a + 2, idx_s0, idx_d0, semi0)
            start_ea(a + 1)
            scatter(idx_d0, rows0)
            start_g(idx_s0, idx_d0, rows0, semg0, semi0)
            finish(idx_s1, idx_d1, rows1, semg1)
            start_idx(a + 3, idx_s1, idx_d1, semi1)
            start_ea(a + 2)
            scatter(idx_d1, rows1)
            return carry

        lax.fori_loop(0, (NFULL - 1) // 2, pair, 0)
        finish(idx_s0, idx_d0, rows0, semg0)
        scatter(idx_d0, rows0)
        pltpu.make_async_copy(gsrc.at[pl.ds(0, CH)], idx_s1, semi1).wait()
        pltpu.make_async_copy(dstv.at[pl.ds(0, CH)], idx_d1, semi1).wait()

        plsc.subcore_barrier()

        def _out(o):
            pltpu.sync_copy(agg_sh.at[pl.ds(o, STG)], rows0)
            pltpu.sync_copy(rows0, out_t.at[c, pl.ds(o, STG)])

        _tile_rows(s, _out)


def _sc_edge_call(pre0, pre1, ea0, ea1, gsrc, dstv):
    zrow = jnp.zeros((STG, H), jnp.float32)
    mesh = plsc.VectorSubcoreMesh(core_axis_name="c", subcore_axis_name="s")
    fn = functools.partial(
        pl.kernel,
        mesh=mesh,
        out_type=[
            jax.ShapeDtypeStruct((B, N, H), jnp.float32),
            jax.ShapeDtypeStruct((B, N, H), jnp.float32),
        ],
        scratch_types=[
            pltpu.VMEM((CH,), jnp.int32),
            pltpu.VMEM((CH,), jnp.int32),
            pltpu.VMEM((CH,), jnp.int32),
            pltpu.VMEM((CH,), jnp.int32),
            pltpu.VMEM((CH, H), jnp.float32),
            pltpu.VMEM((CH, H), jnp.float32),
            pltpu.VMEM((CH, H), jnp.float32),
            pltpu.VMEM_SHARED((N, H), jnp.float32),
            pltpu.SemaphoreType.DMA,
            pltpu.SemaphoreType.DMA,
            pltpu.SemaphoreType.DMA,
            pltpu.SemaphoreType.DMA,
            pltpu.SemaphoreType.DMA,
        ],
    )(_sc_edge_kernel)
    return fn(pre0, pre1, ea0, ea1, gsrc, dstv, zrow)


EPT_D = E // (2 * NT)
NFULL_D = EPT_D // CH
TAIL_D = EPT_D - NFULL_D * CH


def _sc_deg_kernel(dstv, zcnt, ones_hbm, cnt0_out, cnt1_out,
                   idx_d, idx_t, ones_v, stg, stg16, cnt_sh):
    c = lax.axis_index("c")
    s = lax.axis_index("s")
    ebase = c * (E // 2) + s * EPT_D

    pltpu.sync_copy(ones_hbm, ones_v)
    pltpu.sync_copy(zcnt, stg)
    _tile_rows(s, lambda o: pltpu.sync_copy(
        stg, cnt_sh.at[pl.ds(o, STG)]))
    plsc.subcore_barrier()

    def chunk(i, carry):
        e0 = ebase + i * CH
        pltpu.sync_copy(dstv.at[pl.ds(e0, CH)], idx_d)
        pltpu.sync_copy(ones_v, cnt_sh.at[idx_d], add=True)
        return carry

    lax.fori_loop(0, NFULL_D, chunk, 0)
    e0t = ebase + NFULL_D * CH
    pltpu.sync_copy(dstv.at[pl.ds(e0t, TAIL_D)], idx_t)
    pltpu.sync_copy(ones_v.at[pl.ds(0, TAIL_D)],
                    cnt_sh.at[idx_t], add=True)
    plsc.subcore_barrier()

    def _out(o):
        pltpu.sync_copy(cnt_sh.at[pl.ds(o, STG)], stg)

        def row(j, cc):
            stg16[j, pl.ds(0, 16)] = stg[j, pl.ds(0, 16)]
            return cc

        lax.fori_loop(0, STG, row, 0)

        @pl.when(c == 0)
        def _():
            pltpu.sync_copy(stg16, cnt0_out.at[pl.ds(o, STG)])

        @pl.when(c == 1)
        def _():
            pltpu.sync_copy(stg16, cnt1_out.at[pl.ds(o, STG)])

    _tile_rows(s, _out)


def _sc_deg_call(dstv):
    zcnt = jnp.zeros((STG, H), jnp.float32)
    ones_hbm = jnp.ones((CH, H), jnp.float32)
    mesh = plsc.VectorSubcoreMesh(core_axis_name="c", subcore_axis_name="s")
    fn = functools.partial(
        pl.kernel,
        mesh=mesh,
        out_type=[jax.ShapeDtypeStruct((N, 16), jnp.float32),
                  jax.ShapeDtypeStruct((N, 16), jnp.float32)],
        scratch_types=[
            pltpu.VMEM((CH,), jnp.int32),
            pltpu.VMEM((TAIL_D,), jnp.int32),
            pltpu.VMEM((CH, H), jnp.float32),
            pltpu.VMEM((STG, H), jnp.float32),
            pltpu.VMEM((STG, 16), jnp.float32),
            pltpu.VMEM_SHARED((N, H), jnp.float32),
        ],
    )(_sc_deg_kernel)
    return fn(dstv, zcnt, ones_hbm)


def _upd_body(h_ref, s0_ref, s1_ref, cnt0_ref, cnt1_ref,
              w2m0_ref, w2m1_ref, b2m_ref,
              w1uh_ref, w1ua_ref, b1u_ref, w2u_ref, b2u_ref, g_ref, be_ref,
              out_ref):
    hb = h_ref[...]
    agg = jnp.dot(s0_ref[...], w2m0_ref[...], preferred_element_type=jnp.float32)
    agg = agg + jnp.dot(s1_ref[...], w2m1_ref[...], preferred_element_type=jnp.float32)
    deg = cnt0_ref[:, 0:1] + cnt1_ref[:, 0:1]
    agg = agg + deg * b2m_ref[...]
    u = jnp.dot(hb, w1uh_ref[...], preferred_element_type=jnp.float32)
    u = u + jnp.dot(agg, w1ua_ref[...], preferred_element_type=jnp.float32)
    u = u + b1u_ref[...]
    g2 = _gelu_exact(u)
    dh = jnp.dot(g2, w2u_ref[...], preferred_element_type=jnp.float32) + b2u_ref[...]
    y = hb + dh
    mu = jnp.mean(y, axis=-1, keepdims=True)
    var = jnp.mean((y - mu) ** 2, axis=-1, keepdims=True)
    out_ref[...] = (y - mu) * lax.rsqrt(var + 1e-5) * g_ref[...] + be_ref[...]


def _upd_call(h_flat, s0, s1, cnt0, cnt1, w2m, b2m, w1u, b1u, w2u, b2u,
              gamma, beta):
    blk = 400
    grid = (B * N) // blk
    nblk = N // blk
    full = lambda shape: pl.BlockSpec(shape, lambda i: tuple(0 for _ in shape))
    return pl.pallas_call(
        _upd_body,
        grid=(grid,),
        in_specs=[
            pl.BlockSpec((blk, D), lambda i: (i, 0)),
            pl.BlockSpec((blk, H), lambda i: (i, 0)),
            pl.BlockSpec((blk, H), lambda i: (i, 0)),
            pl.BlockSpec((blk, 16), lambda i: (i % nblk, 0)),
            pl.BlockSpec((blk, 16), lambda i: (i % nblk, 0)),
            full((H, D)), full((H, D)), full((1, D)),
            full((D, D)), full((D, D)), full((1, D)),
            full((D, D)), full((1, D)), full((1, D)), full((1, D)),
        ],
        out_specs=pl.BlockSpec((blk, D), lambda i: (i, 0)),
        out_shape=jax.ShapeDtypeStruct((B * N, D), jnp.float32),
    )(h_flat, s0, s1, cnt0, cnt1, w2m[:H], w2m[H:], b2m.reshape(1, D),
      w1u[:D], w1u[D:], b1u.reshape(1, D), w2u, b2u.reshape(1, D),
      gamma.reshape(1, D), beta.reshape(1, D))


def kernel(h, edge_index, edge_attr, W1m, b1m, W2m, b2m,
           W1u, b1u, W2u, b2u, gamma, beta):
    h_flat = h.reshape(B * N, D)
    src = edge_index[0]
    dst = edge_index[1]
    offs = (jnp.arange(B, dtype=jnp.int32) * N)[:, None]
    pad = jnp.zeros((CH,), jnp.int32)
    gsrc = jnp.concatenate([(src[None, :] + offs).reshape(-1), pad])
    dst_p = jnp.concatenate([dst, pad])

    cnt0, cnt1 = _sc_deg_call(dst)
    pre0, pre1 = _pre_call(h_flat, W1m[:D], b1m.reshape(1, D))
    ea0, ea1 = _ea_call(edge_attr, W1m[D:])

    s0, s1 = _sc_edge_call(pre0, pre1, ea0, ea1, gsrc, dst_p)

    out = _upd_call(h_flat, s0.reshape(B * N, H), s1.reshape(B * N, H),
                    cnt0, cnt1, W2m, b2m, W1u, b1u, W2u, b2u,
                    gamma, beta)
    return out.reshape(B, N, D)
